# baseline (speedup 1.0000x reference)
import functools

import jax
import jax.numpy as jnp
from jax import lax
from jax.experimental import pallas as pl
from jax.experimental.pallas import tpu as pltpu
from jax.experimental.pallas import tpu_sc as plsc

_ROWS = 16384
_ATOMS = 200
_LANES = 16

_info = plsc.get_sparse_core_info()
_NC, _NS = _info.num_cores, _info.num_subcores
_NW = _NC * _NS
_RPW = _ROWS // _NW
_CH = 128
_NCHUNK = _RPW // _CH
_GRP = _CH // _LANES


def _body(sp_hbm, en_hbm, tab_hbm, spo_hbm, eno_hbm,
          tab_v, en_v, out_v,
          b0, b1, b2, b3, si0, si1, si2, si3, so0, so1, so2, so3):
    wid = lax.axis_index("s") * _NC + lax.axis_index("c")
    base = wid * _RPW
    bufs = (b0, b1, b2, b3)
    isems = (si0, si1, si2, si3)
    osems = (so0, so1, so2, so3)

    in_cps = [
        pltpu.async_copy(
            sp_hbm.at[:, pl.ds(base + c * _CH, _CH)], bufs[c], isems[c])
        for c in range(_NCHUNK)
    ]
    pltpu.sync_copy(tab_hbm, tab_v.at[pl.ds(0, 7)])
    pltpu.sync_copy(en_hbm.at[pl.ds(base, _RPW)], en_v)

    lane = lax.iota(jnp.int32, _LANES)
    tv = jnp.where(lane < 7, tab_v[...], 0.0)
    zero = jnp.zeros((_LANES,), jnp.float32)

    def lookup(s):
        return lax.gather(
            tv, (s & 15)[:, None],
            dimension_numbers=lax.GatherDimensionNumbers(
                offset_dims=(), collapsed_slice_dims=(0,),
                start_index_map=(0,)),
            slice_sizes=(1,),
            mode=lax.GatherScatterMode.PROMISE_IN_BOUNDS)

    out_cps = []
    for c in range(_NCHUNK):
        in_cps[c].wait()
        buf = bufs[c]
        out_cps.append(pltpu.async_copy(
            buf, spo_hbm.at[:, pl.ds(base + c * _CH, _CH)], osems[c]))

        def group(g, carry):
            c0 = pl.multiple_of(g * _LANES, _LANES)
            sl = pl.ds(c0, _LANES)

            def step(i, c2):
                a0, a1, a2, a3 = c2
                r = i * 4
                return (a0 + lookup(buf[r, sl]),
                        a1 + lookup(buf[r + 1, sl]),
                        a2 + lookup(buf[r + 2, sl]),
                        a3 + lookup(buf[r + 3, sl]))

            a0, a1, a2, a3 = lax.fori_loop(
                0, _ATOMS // 4, step, (zero, zero, zero, zero), unroll=10)
            off = pl.multiple_of(c * _CH + g * _LANES, _LANES)
            out_v[pl.ds(off, _LANES)] = (a0 + a1) + (a2 + a3)
            return carry

        lax.fori_loop(0, _GRP, group, 0)

    for i in range(_RPW // _LANES):
        sl = pl.ds(i * _LANES, _LANES)
        out_v[sl] = out_v[sl] + en_v[sl]
    pltpu.sync_copy(out_v, eno_hbm.at[pl.ds(base, _RPW)])
    for cp in out_cps:
        cp.wait()


_sc_call = functools.partial(
    pl.kernel,
    mesh=plsc.VectorSubcoreMesh(core_axis_name="c", subcore_axis_name="s"),
    compiler_params=pltpu.CompilerParams(
        needs_layout_passes=False, skip_device_barrier=True,
        disable_bounds_checks=True, disable_semaphore_checks=True),
    out_type=(
        jax.ShapeDtypeStruct((_ATOMS, _ROWS), jnp.int32),
        jax.ShapeDtypeStruct((_ROWS,), jnp.float32),
    ),
    scratch_types=(
        [pltpu.VMEM((_LANES,), jnp.float32),
         pltpu.VMEM((_RPW,), jnp.float32),
         pltpu.VMEM((_RPW,), jnp.float32)]
        + [pltpu.VMEM((_ATOMS, _CH), jnp.int32)] * _NCHUNK
        + [pltpu.SemaphoreType.DMA] * (2 * _NCHUNK)
    ),
)(_body)


def kernel(species, energies, self_energies):
    spt = jnp.asarray(species, jnp.int32).T
    spo_t, en_out = _sc_call(spt, energies.astype(jnp.float32),
                             self_energies.astype(jnp.float32))
    return (spo_t.T, en_out)
